# Initial kernel scaffold; baseline (speedup 1.0000x reference)
import jax
import jax.numpy as jnp
from jax.experimental import pallas as pl


def kernel(energies, concentrations, receptor_indices, epsilon_units):
    raise NotImplementedError("write your pallas kernel here")


# speedup vs baseline: 8.7391x; 8.7391x over previous
import jax
import jax.numpy as jnp
from jax.experimental import pallas as pl
from jax.experimental.pallas import tpu as pltpu


def _split_hi_lo(v):
    hi = v.astype(jnp.bfloat16)
    lo = (v - hi.astype(jnp.float32)).astype(jnp.bfloat16)
    return hi, lo


def _mwc_kernel(
    eo_ref, ec_ref, c_ref, idx_ref, eps_ref, out_ref,
    ph_scr, plo_scr, dh_scr, dlo_scr, s_scr, er_scr,
):
    ib = pl.program_id(0)
    ir = pl.program_id(1)
    n_units = eo_ref.shape[1]
    br = out_ref.shape[1]

    @pl.when(ir == 0)
    def _():
        c = c_ref[...]
        eo = eo_ref[...]
        ec = ec_ref[...]
        p = jnp.log1p(c * jnp.exp(-ec)) - jnp.log1p(c * jnp.exp(-eo))
        ph_scr[...], plo_scr[...] = _split_hi_lo(p)
        dh_scr[...], dlo_scr[...] = _split_hi_lo(eo - ec)

    @pl.when(ib == 0)
    def _():
        idx = idx_ref[...]
        u_iota = jax.lax.broadcasted_iota(jnp.int32, (n_units, br), 0)
        s = jnp.zeros((n_units, br), jnp.float32)
        for k in range(idx_ref.shape[0]):
            s = s + jnp.where(u_iota == idx[k : k + 1, :], 1.0, 0.0)
        sb = s.astype(jnp.bfloat16)
        s_scr[:, pl.ds(ir * br, br)] = sb
        eh, elo = _split_hi_lo(eps_ref[...])
        er = jnp.dot(eh, sb, preferred_element_type=jnp.float32) + jnp.dot(
            elo, sb, preferred_element_type=jnp.float32
        )
        er_scr[0:1, pl.ds(ir * br, br)] = er

    sb = s_scr[:, pl.ds(ir * br, br)]
    x = jnp.dot(ph_scr[...], sb, preferred_element_type=jnp.float32) + jnp.dot(
        plo_scr[...], sb, preferred_element_type=jnp.float32
    )
    sd = jnp.dot(dh_scr[...], sb, preferred_element_type=jnp.float32) + jnp.dot(
        dlo_scr[...], sb, preferred_element_type=jnp.float32
    )
    er = er_scr[0:1, pl.ds(ir * br, br)]

    L = jnp.exp(-er)
    p_min = 1.0 / (1.0 + L)
    p_c = 1.0 / (1.0 + L * jnp.exp(x))
    p_max = 1.0 / (1.0 + L * jnp.exp(sd))
    denom = p_max - p_min
    norm = (p_c - p_min) / (denom + 1e-8)
    norm = jnp.where(denom > 1e-6, norm, 0.0)
    out_ref[...] = jnp.clip(norm, 0.0, 1.0)


@jax.jit
def kernel(energies, concentrations, receptor_indices, epsilon_units):
    b, u, _ = energies.shape
    r, k = receptor_indices.shape
    bb = 512
    br = 1024
    nb = b // bb
    nr = r // br

    e = jnp.transpose(energies, (2, 0, 1))
    eo, ec = e[0], e[1]
    c2 = concentrations.reshape(b, 1)
    idxt = receptor_indices.T
    eps2 = epsilon_units.reshape(1, u)

    return pl.pallas_call(
        _mwc_kernel,
        grid=(nb, nr),
        in_specs=[
            pl.BlockSpec((bb, u), lambda ib, ir: (ib, 0)),
            pl.BlockSpec((bb, u), lambda ib, ir: (ib, 0)),
            pl.BlockSpec((bb, 1), lambda ib, ir: (ib, 0)),
            pl.BlockSpec((k, br), lambda ib, ir: (0, ir)),
            pl.BlockSpec((1, u), lambda ib, ir: (0, 0)),
        ],
        out_specs=pl.BlockSpec((bb, br), lambda ib, ir: (ib, ir)),
        out_shape=jax.ShapeDtypeStruct((b, r), jnp.float32),
        scratch_shapes=[
            pltpu.VMEM((bb, u), jnp.bfloat16),
            pltpu.VMEM((bb, u), jnp.bfloat16),
            pltpu.VMEM((bb, u), jnp.bfloat16),
            pltpu.VMEM((bb, u), jnp.bfloat16),
            pltpu.VMEM((u, r), jnp.bfloat16),
            pltpu.VMEM((8, r), jnp.float32),
        ],
    )(eo, ec, c2, idxt, eps2)
